# baseline (speedup 1.0000x reference)
import jax
import jax.numpy as jnp
from jax import lax
from jax.experimental import pallas as pl

_H = 8
_NK = 256
_K = 8
_TN = 128


def _top8_vals(vals):
    tv = []
    for _ in range(_K):
        m = jnp.max(vals, axis=1, keepdims=True)
        tv.append(m)
        vals = jnp.where(vals == m, -jnp.inf, vals)
    return tv


def _main_body(x_ref, wq_ref, k_ref, dn_ref, up_ref, o_ref):
    dk = k_ref.shape[-1]
    xt = x_ref[...]
    q = lax.dot_general(xt, wq_ref[...], (((1,), (1,)), ((), ())),
                        preferred_element_type=jnp.float32)
    hfull = lax.dot_general(xt, dn_ref[...], (((1,), (1,)), ((), ())),
                            precision=lax.Precision.HIGHEST,
                            preferred_element_type=jnp.float32)
    act64 = hfull * (1.0 / (1.0 + jnp.exp(-hfull)))

    c64 = jnp.zeros((_TN, 64), jnp.float32)
    for h in range(_H):
        qx = q[:, h * dk:(h + 1) * dk]
        qy = q[:, (_H + h) * dk:(_H + h + 1) * dk]
        sx_all = lax.dot_general(qx, k_ref[0, h], (((1,), (1,)), ((), ())),
                                 preferred_element_type=jnp.float32)
        sy_all = lax.dot_general(qy, k_ref[1, h], (((1,), (1,)), ((), ())),
                                 preferred_element_type=jnp.float32)
        sx = _top8_vals(sx_all)
        sy = _top8_vals(sy_all)
        sy_row = jnp.concatenate(sy, axis=1)
        grid = jnp.concatenate([sx[i] + sy_row for i in range(_K)],
                               axis=1)
        v = grid
        r = jnp.full((_TN, 64), -jnp.inf, jnp.float32)
        m0 = None
        for _ in range(_K):
            m = jnp.max(v, axis=1, keepdims=True)
            if m0 is None:
                m0 = m
            hit = v == m
            r = jnp.where(hit, v, r)
            v = jnp.where(hit, -jnp.inf, v)
        acc = jnp.exp(r - m0)
        z = jnp.sum(acc, axis=1, keepdims=True)
        c64 = c64 + (acc / z) * act64
    o_ref[...] = lax.dot_general(c64, up_ref[...], (((1,), (0,)), ((), ())),
                                 precision=lax.Precision.HIGHEST,
                                 preferred_element_type=jnp.float32)


def kernel(x, W_q, keys, down_embed, up_embed):
    b, n, d = x.shape
    dk = d // 2
    x2 = x.reshape(b * n, d)
    keys_t = jnp.transpose(keys, (2, 0, 1, 3))

    dn64 = down_embed[:64]
    up64 = up_embed[:64]

    out = pl.pallas_call(
        _main_body,
        grid=(b * n // _TN,),
        in_specs=[
            pl.BlockSpec((_TN, d), lambda i: (i, 0)),
            pl.BlockSpec((2 * _H * dk, d), lambda i: (0, 0)),
            pl.BlockSpec((2, _H, _NK, dk), lambda i: (0, 0, 0, 0)),
            pl.BlockSpec((64, d), lambda i: (0, 0)),
            pl.BlockSpec((64, d), lambda i: (0, 0)),
        ],
        out_specs=pl.BlockSpec((_TN, d), lambda i: (i, 0)),
        out_shape=jax.ShapeDtypeStruct((b * n, d), jnp.float32),
    )(x2, W_q, keys_t, dn64, up64)
    return out.reshape(b, n, d)


# speedup vs baseline: 1.2091x; 1.2091x over previous
import jax
import jax.numpy as jnp
from jax import lax
from jax.experimental import pallas as pl

_H = 8
_NK = 256
_K = 8
_TN = 256


def _top8_vals(vals):
    tv = []
    for _ in range(_K):
        m = jnp.max(vals, axis=1, keepdims=True)
        tv.append(m)
        vals = jnp.where(vals == m, -jnp.inf, vals)
    return tv


def _main_body(x_ref, wq_ref, k_ref, dn_ref, up_ref, o_ref):
    dk = k_ref.shape[-1]
    xt = x_ref[...]
    q = lax.dot_general(xt, wq_ref[...], (((1,), (1,)), ((), ())),
                        preferred_element_type=jnp.float32)
    hfull = lax.dot_general(xt, dn_ref[...], (((1,), (1,)), ((), ())),
                            precision=lax.Precision.HIGHEST,
                            preferred_element_type=jnp.float32)
    act64 = hfull * (1.0 / (1.0 + jnp.exp(-hfull)))

    c64 = jnp.zeros((_TN, 64), jnp.float32)
    for h in range(_H):
        qx = q[:, h * dk:(h + 1) * dk]
        qy = q[:, (_H + h) * dk:(_H + h + 1) * dk]
        sx_all = lax.dot_general(qx, k_ref[0, h], (((1,), (1,)), ((), ())),
                                 preferred_element_type=jnp.float32)
        sy_all = lax.dot_general(qy, k_ref[1, h], (((1,), (1,)), ((), ())),
                                 preferred_element_type=jnp.float32)
        sx = _top8_vals(sx_all)
        sy = _top8_vals(sy_all)
        sy_row = jnp.concatenate(sy, axis=1)
        grid = jnp.concatenate([sx[i] + sy_row for i in range(_K)],
                               axis=1)
        v = grid
        r = jnp.full((_TN, 64), -jnp.inf, jnp.float32)
        m0 = None
        for _ in range(_K):
            m = jnp.max(v, axis=1, keepdims=True)
            if m0 is None:
                m0 = m
            hit = v == m
            r = jnp.where(hit, v, r)
            v = jnp.where(hit, -jnp.inf, v)
        acc = jnp.exp(r - m0)
        z = jnp.sum(acc, axis=1, keepdims=True)
        c64 = c64 + (acc / z) * act64
    o_ref[...] = lax.dot_general(c64, up_ref[...], (((1,), (0,)), ((), ())),
                                 precision=lax.Precision.HIGHEST,
                                 preferred_element_type=jnp.float32)


def kernel(x, W_q, keys, down_embed, up_embed):
    b, n, d = x.shape
    dk = d // 2
    x2 = x.reshape(b * n, d)
    keys_t = jnp.transpose(keys, (2, 0, 1, 3))

    dn64 = down_embed[:64]
    up64 = up_embed[:64]

    out = pl.pallas_call(
        _main_body,
        grid=(b * n // _TN,),
        in_specs=[
            pl.BlockSpec((_TN, d), lambda i: (i, 0)),
            pl.BlockSpec((2 * _H * dk, d), lambda i: (0, 0)),
            pl.BlockSpec((2, _H, _NK, dk), lambda i: (0, 0, 0, 0)),
            pl.BlockSpec((64, d), lambda i: (0, 0)),
            pl.BlockSpec((64, d), lambda i: (0, 0)),
        ],
        out_specs=pl.BlockSpec((_TN, d), lambda i: (i, 0)),
        out_shape=jax.ShapeDtypeStruct((b * n, d), jnp.float32),
    )(x2, W_q, keys_t, dn64, up64)
    return out.reshape(b, n, d)


# speedup vs baseline: 1.7252x; 1.4269x over previous
import jax
import jax.numpy as jnp
from jax import lax
from jax.experimental import pallas as pl

_H = 8
_NK = 256
_K = 8
_TN = 512


def _top8_scatter(vals, masks, out):
    for i in range(_K):
        m = jnp.max(vals, axis=1, keepdims=True)
        out = jnp.where(masks[i], m, out)
        vals = jnp.where(vals == m, -jnp.inf, vals)
    return out


def _main_body(x_ref, wq_ref, k_ref, dn_ref, up_ref, o_ref):
    dk = k_ref.shape[-1]
    xt = x_ref[...]
    q = lax.dot_general(xt, wq_ref[...], (((1,), (1,)), ((), ())),
                        preferred_element_type=jnp.float32)
    hfull = lax.dot_general(xt, dn_ref[...], (((1,), (1,)), ((), ())),
                            precision=lax.Precision.HIGHEST,
                            preferred_element_type=jnp.float32)
    act64 = hfull * (1.0 / (1.0 + jnp.exp(-hfull)))

    iota_64 = lax.broadcasted_iota(jnp.int32, (_TN, 64), 1)
    xmasks = [(iota_64 // _K) == i for i in range(_K)]
    ymasks = [(iota_64 % _K) == j for j in range(_K)]
    zero64 = jnp.zeros((_TN, 64), jnp.float32)

    c64 = jnp.zeros((_TN, 64), jnp.float32)
    for h in range(_H):
        qx = q[:, h * dk:(h + 1) * dk]
        qy = q[:, (_H + h) * dk:(_H + h + 1) * dk]
        sx_all = lax.dot_general(qx, k_ref[0, h], (((1,), (1,)), ((), ())),
                                 preferred_element_type=jnp.float32)
        sy_all = lax.dot_general(qy, k_ref[1, h], (((1,), (1,)), ((), ())),
                                 preferred_element_type=jnp.float32)
        sxg = _top8_scatter(sx_all, xmasks, zero64)
        syg = _top8_scatter(sy_all, ymasks, zero64)
        grid = sxg + syg
        v = grid
        r = jnp.full((_TN, 64), -jnp.inf, jnp.float32)
        m0 = None
        for _ in range(_K):
            m = jnp.max(v, axis=1, keepdims=True)
            if m0 is None:
                m0 = m
            hit = v == m
            r = jnp.where(hit, v, r)
            v = jnp.where(hit, -jnp.inf, v)
        acc = jnp.exp(r - m0)
        z = jnp.sum(acc, axis=1, keepdims=True)
        c64 = c64 + (acc / z) * act64
    o_ref[...] = lax.dot_general(c64, up_ref[...], (((1,), (0,)), ((), ())),
                                 precision=lax.Precision.HIGHEST,
                                 preferred_element_type=jnp.float32)


def kernel(x, W_q, keys, down_embed, up_embed):
    b, n, d = x.shape
    dk = d // 2
    x2 = x.reshape(b * n, d)
    keys_t = jnp.transpose(keys, (2, 0, 1, 3))

    dn64 = down_embed[:64]
    up64 = up_embed[:64]

    out = pl.pallas_call(
        _main_body,
        grid=(b * n // _TN,),
        in_specs=[
            pl.BlockSpec((_TN, d), lambda i: (i, 0)),
            pl.BlockSpec((2 * _H * dk, d), lambda i: (0, 0)),
            pl.BlockSpec((2, _H, _NK, dk), lambda i: (0, 0, 0, 0)),
            pl.BlockSpec((64, d), lambda i: (0, 0)),
            pl.BlockSpec((64, d), lambda i: (0, 0)),
        ],
        out_specs=pl.BlockSpec((_TN, d), lambda i: (i, 0)),
        out_shape=jax.ShapeDtypeStruct((b * n, d), jnp.float32),
    )(x2, W_q, keys_t, dn64, up64)
    return out.reshape(b, n, d)


# speedup vs baseline: 1.7284x; 1.0019x over previous
import jax
import jax.numpy as jnp
from jax import lax
from jax.experimental import pallas as pl

_H = 8
_NK = 256
_K = 8
_TN = 512


def _top8_scatter(vals, masks, out):
    for i in range(_K):
        m = jnp.max(vals, axis=1, keepdims=True)
        out = jnp.where(masks[i], m, out)
        vals = jnp.where(vals == m, -jnp.inf, vals)
    return out


def _main_body(x_ref, wq_ref, k_ref, dn_ref, up_ref, o_ref):
    dk = k_ref.shape[-1]
    xt = x_ref[...]
    q = lax.dot_general(xt, wq_ref[...], (((1,), (1,)), ((), ())),
                        preferred_element_type=jnp.float32)
    hfull = lax.dot_general(xt, dn_ref[...], (((1,), (1,)), ((), ())),
                            precision=lax.Precision.HIGHEST,
                            preferred_element_type=jnp.float32)
    act64 = hfull * (1.0 / (1.0 + jnp.exp(-hfull)))

    iota_64 = lax.broadcasted_iota(jnp.int32, (_TN, 64), 1)
    xmasks = [(iota_64 // _K) == i for i in range(_K)]
    ymasks = [(iota_64 % _K) == j for j in range(_K)]
    zero64 = jnp.zeros((_TN, 64), jnp.float32)

    c64 = jnp.zeros((_TN, 64), jnp.float32)
    for h in range(_H):
        qx = q[:, h * dk:(h + 1) * dk]
        qy = q[:, (_H + h) * dk:(_H + h + 1) * dk]
        sx_all = lax.dot_general(qx, k_ref[0, h], (((1,), (1,)), ((), ())),
                                 preferred_element_type=jnp.float32)
        sy_all = lax.dot_general(qy, k_ref[1, h], (((1,), (1,)), ((), ())),
                                 preferred_element_type=jnp.float32)
        sxg = _top8_scatter(sx_all, xmasks, zero64)
        syg = _top8_scatter(sy_all, ymasks, zero64)
        grid = sxg + syg
        v = grid
        r = jnp.full((_TN, 64), -jnp.inf, jnp.float32)
        m0 = None
        for _ in range(_K):
            m = jnp.max(v, axis=1, keepdims=True)
            if m0 is None:
                m0 = m
            hit = v == m
            r = jnp.where(hit, v, r)
            v = jnp.where(hit, -jnp.inf, v)
        acc = jnp.exp(r - m0)
        z = jnp.sum(acc, axis=1, keepdims=True)
        c64 = c64 + acc / z
    c64 = c64 * act64
    o_ref[...] = lax.dot_general(c64, up_ref[...], (((1,), (0,)), ((), ())),
                                 precision=lax.Precision.HIGHEST,
                                 preferred_element_type=jnp.float32)


def kernel(x, W_q, keys, down_embed, up_embed):
    b, n, d = x.shape
    dk = d // 2
    x2 = x.reshape(b * n, d)
    keys_t = jnp.transpose(keys, (2, 0, 1, 3))

    dn64 = down_embed[:64]
    up64 = up_embed[:64]

    out = pl.pallas_call(
        _main_body,
        grid=(b * n // _TN,),
        in_specs=[
            pl.BlockSpec((_TN, d), lambda i: (i, 0)),
            pl.BlockSpec((2 * _H * dk, d), lambda i: (0, 0)),
            pl.BlockSpec((2, _H, _NK, dk), lambda i: (0, 0, 0, 0)),
            pl.BlockSpec((64, d), lambda i: (0, 0)),
            pl.BlockSpec((64, d), lambda i: (0, 0)),
        ],
        out_specs=pl.BlockSpec((_TN, d), lambda i: (i, 0)),
        out_shape=jax.ShapeDtypeStruct((b * n, d), jnp.float32),
    )(x2, W_q, keys_t, dn64, up64)
    return out.reshape(b, n, d)
